# baseline (speedup 1.0000x reference)
import jax
import jax.numpy as jnp
import numpy as np
from jax.experimental import pallas as pl
from jax.experimental.pallas import tpu as pltpu

_ALPHA = 0.3
_BETA = 0.3
_B = 256
_R = 4

with jax.default_device(jax.local_devices(backend="cpu")[0]):
    _PERM = np.asarray(
        jax.random.permutation(jax.random.fold_in(jax.random.key(42), 1), _B)
    ).astype(np.int64)

_PMAT = np.zeros((_B, _B), dtype=np.float32)
_PMAT[np.arange(_B), _PERM] = 1.0
_PERM32 = _PERM.astype(np.int32)


def _mix_body(idx_ref, lamb_ref, xd_ref, *refs):
    lam = lamb_ref[0]
    xp = refs[:_R]
    ox_ref = refs[_R]
    for r in range(_R):
        ox_ref[r] = lam * xd_ref[r] + (1.0 - lam) * xp[r][0]


def _y_body(m_ref, y_ref, oy_ref):
    oy_ref[...] = jnp.dot(m_ref[...], y_ref[...],
                          preferred_element_type=jnp.float32)


def kernel(x, y):
    kl = jax.random.fold_in(jax.random.key(42), 0)
    lamb = jax.random.beta(kl, _ALPHA, _BETA, dtype=jnp.float32)

    B, C, H, W = x.shape
    D = C * H * W
    S = D // 128
    xf = x.reshape(B, S, 128)
    L = y.shape[1]

    def pmap(r):
        return lambda k, idx, lam: (idx[_R * k + r], 0, 0)

    grid_spec = pltpu.PrefetchScalarGridSpec(
        num_scalar_prefetch=2,
        grid=(B // _R,),
        in_specs=[pl.BlockSpec((_R, S, 128), lambda k, idx, lam: (k, 0, 0))]
        + [pl.BlockSpec((1, S, 128), pmap(r)) for r in range(_R)],
        out_specs=pl.BlockSpec((_R, S, 128), lambda k, idx, lam: (k, 0, 0)),
    )

    mixed_xf = pl.pallas_call(
        _mix_body,
        grid_spec=grid_spec,
        out_shape=jax.ShapeDtypeStruct((B, S, 128), jnp.float32),
    )(jnp.asarray(_PERM32), lamb.reshape(1), *([xf] * (1 + _R)))

    mmat = lamb * jnp.eye(B, dtype=jnp.float32) \
        + (1.0 - lamb) * jnp.asarray(_PMAT)
    mixed_y = pl.pallas_call(
        _y_body,
        out_shape=jax.ShapeDtypeStruct((B, L), jnp.float32),
    )(mmat, y)

    return (mixed_xf.reshape(B, C, H, W), mixed_y)


# speedup vs baseline: 1.1324x; 1.1324x over previous
import functools

import jax
import jax.numpy as jnp
import numpy as np
from jax import lax
from jax.experimental import pallas as pl
from jax.experimental.pallas import tpu as pltpu
from jax.experimental.pallas import tpu_sc as plsc

_ALPHA = 0.3
_BETA = 0.3
_B = 256

with jax.default_device(jax.local_devices(backend="cpu")[0]):
    _PERM = np.asarray(
        jax.random.permutation(jax.random.fold_in(jax.random.key(42), 1), _B)
    ).astype(np.int64)

_PMAT = np.zeros((_B, _B), dtype=np.float32)
_PMAT[np.arange(_B), _PERM] = 1.0

_NC = 2
_NW = 32
_Q = 3072
_QPW = _Q // _NW
_SL = 56
_LN = 224

_qb = np.arange(_Q) // 12
_qr = np.arange(_Q) % 12
_PQ = (_PERM[_qb] * 12 + _qr).astype(np.int32)
_IDXP = np.repeat(_PQ.reshape(_NW, _QPW, 1), 16, axis=2)


@functools.partial(
    pl.kernel,
    out_type=jax.ShapeDtypeStruct((_Q, _SL, _LN), jnp.float32),
    mesh=plsc.VectorSubcoreMesh(core_axis_name="c", subcore_axis_name="s"),
    compiler_params=pltpu.CompilerParams(use_tc_tiling_on_sc=True),
    scratch_types=[
        pltpu.VMEM((_QPW, 16), jnp.int32),
        pltpu.VMEM((16,), jnp.float32),
        pltpu.VMEM((2, _SL, _LN), jnp.float32),
        pltpu.VMEM((2, _SL, _LN), jnp.float32),
        pltpu.VMEM((2, _SL, _LN), jnp.float32),
        pltpu.SemaphoreType.DMA,
        pltpu.SemaphoreType.DMA,
        pltpu.SemaphoreType.DMA,
    ],
)
def _sc_mix(xq_hbm, idxp_hbm, lamb_hbm, out_hbm,
            idxp_v, lamb_v, dbuf, pbuf, obuf, semd, semp, semo):
    wid = lax.axis_index("s") * _NC + lax.axis_index("c")
    base = wid * _QPW
    pltpu.sync_copy(idxp_hbm.at[wid], idxp_v)
    pltpu.sync_copy(lamb_hbm, lamb_v)
    lam = lamb_v[...]
    om = 1.0 - lam

    def perm_row(j):
        return idxp_v[j, :][0]

    def start_gather(j, b):
        pltpu.async_copy(xq_hbm.at[base + j], dbuf.at[b], semd)
        pltpu.async_copy(xq_hbm.at[perm_row(j)], pbuf.at[b], semp)

    def wait_gather(b):
        pltpu.make_async_copy(xq_hbm.at[0], dbuf.at[b], semd).wait()
        pltpu.make_async_copy(xq_hbm.at[0], pbuf.at[b], semp).wait()

    def wait_scatter(b):
        pltpu.make_async_copy(obuf.at[b], out_hbm.at[0], semo).wait()

    def compute(b):
        def vloop(s, carry):
            for v in range(_LN // 16):
                off = v * 16
                obuf[b, s, pl.ds(off, 16)] = (
                    lam * dbuf[b, s, pl.ds(off, 16)]
                    + om * pbuf[b, s, pl.ds(off, 16)]
                )
            return carry

        lax.fori_loop(0, _SL, vloop, 0)

    def start_scatter(j, b):
        pltpu.async_copy(obuf.at[b], out_hbm.at[base + j], semo)

    start_gather(0, 0)
    start_gather(1, 1)
    for b in range(2):
        wait_gather(b)
        compute(b)
        start_scatter(b, b)
        start_gather(b + 2, b)

    def outer(kk, carry):
        for b in range(2):
            j = 2 * kk + b
            wait_gather(b)
            wait_scatter(b)
            compute(b)
            start_scatter(j, b)

            @pl.when(j + 2 < _QPW)
            def _():
                start_gather(j + 2, b)
        return carry

    lax.fori_loop(1, _QPW // 2, outer, 0)
    wait_scatter(0)
    wait_scatter(1)


def _y_body(m_ref, y_ref, oy_ref):
    oy_ref[...] = jnp.dot(m_ref[...], y_ref[...],
                          preferred_element_type=jnp.float32)


def kernel(x, y):
    kl = jax.random.fold_in(jax.random.key(42), 0)
    lamb = jax.random.beta(kl, _ALPHA, _BETA, dtype=jnp.float32)

    B, C, H, W = x.shape
    xq = x.reshape(_Q, _SL, _LN)
    L = y.shape[1]

    mixed_xq = _sc_mix(xq, jnp.asarray(_IDXP),
                       jnp.full((16,), lamb, jnp.float32))

    mmat = lamb * jnp.eye(B, dtype=jnp.float32) \
        + (1.0 - lamb) * jnp.asarray(_PMAT)
    mixed_y = pl.pallas_call(
        _y_body,
        out_shape=jax.ShapeDtypeStruct((B, L), jnp.float32),
    )(mmat, y)

    return (mixed_xq.reshape(B, C, H, W), mixed_y)
